# speedup vs baseline: 2.1818x; 2.1818x over previous
import functools

import jax
import jax.numpy as jnp
from jax import lax
from jax.experimental import pallas as pl
from jax.experimental.pallas import tpu as pltpu
from jax.experimental.pallas import tpu_sc as plsc

N_POSITION = 8192
D_MODEL = 1024
BATCH = 4
SEQ = 8192

NC, NS = 2, 16
NW = NC * NS
B_TOTAL = BATCH * SEQ
BPW = B_TOTAL // NW
R = 64
NCHUNK = BPW // R


@jax.jit
def _sc_gather(x_r, table):
    mesh = plsc.VectorSubcoreMesh(core_axis_name="c", subcore_axis_name="s")

    @functools.partial(
        pl.kernel,
        mesh=mesh,
        out_type=jax.ShapeDtypeStruct((B_TOTAL, D_MODEL), jnp.float32),
        scratch_types=[
            pltpu.VMEM((NCHUNK, R), jnp.int32),
            pltpu.VMEM((R, D_MODEL), jnp.float32),
            pltpu.SemaphoreType.DMA,
        ],
    )
    def k(x_hbm, table_hbm, out_hbm, idx_v, rows_v, sem):
        wid = lax.axis_index("s") * NC + lax.axis_index("c")
        base = wid * BPW
        pltpu.sync_copy(x_hbm.at[wid], idx_v)

        def body(c, carry):
            pltpu.async_copy(table_hbm.at[idx_v.at[c]], rows_v, sem).wait()
            pltpu.sync_copy(rows_v, out_hbm.at[pl.ds(base + c * R, R)])
            return carry

        lax.fori_loop(0, NCHUNK, body, 0)

    return k(x_r, table)


def kernel(x, table):
    x_r = x.reshape(NW, NCHUNK, R)
    out = _sc_gather(x_r, table)
    return out.reshape(BATCH, SEQ, D_MODEL)


# speedup vs baseline: 2.3005x; 1.0544x over previous
import functools

import jax
import jax.numpy as jnp
from jax import lax
from jax.experimental import pallas as pl
from jax.experimental.pallas import tpu as pltpu
from jax.experimental.pallas import tpu_sc as plsc

N_POSITION = 8192
D_MODEL = 1024
BATCH = 4
SEQ = 8192

NC, NS = 2, 16
NW = NC * NS
B_TOTAL = BATCH * SEQ
BPW = B_TOTAL // NW
R = 32
NCHUNK = BPW // R
NGRP = NCHUNK // 2


@jax.jit
def _sc_gather(x_r, table):
    mesh = plsc.VectorSubcoreMesh(core_axis_name="c", subcore_axis_name="s")

    @functools.partial(
        pl.kernel,
        mesh=mesh,
        out_type=jax.ShapeDtypeStruct((B_TOTAL, D_MODEL), jnp.float32),
        scratch_types=[
            pltpu.VMEM((NCHUNK, R), jnp.int32),
            pltpu.VMEM((R, D_MODEL), jnp.float32),
            pltpu.VMEM((R, D_MODEL), jnp.float32),
            pltpu.SemaphoreType.DMA,
            pltpu.SemaphoreType.DMA,
            pltpu.SemaphoreType.DMA,
            pltpu.SemaphoreType.DMA,
        ],
    )
    def k(x_hbm, table_hbm, out_hbm, idx_v, rows0, rows1,
          gsem0, gsem1, wsem0, wsem1):
        wid = lax.axis_index("s") * NC + lax.axis_index("c")
        base = wid * BPW
        pltpu.sync_copy(x_hbm.at[wid], idx_v)

        bufs = (rows0, rows1)
        gsems = (gsem0, gsem1)
        wsems = (wsem0, wsem1)

        def gather(c, b):
            return pltpu.make_async_copy(
                table_hbm.at[idx_v.at[c]], bufs[b], gsems[b])

        def write(c, b):
            return pltpu.make_async_copy(
                bufs[b], out_hbm.at[pl.ds(base + c * R, R)], wsems[b])

        gather(0, 0).start()

        def body(g, carry):
            for b in range(2):
                c = 2 * g + b
                o = 1 - b
                gather(c, b).wait()

                @pl.when(c + 1 < NCHUNK)
                def _():
                    @pl.when(c >= 1)
                    def _():
                        write(c - 1, o).wait()
                    gather(c + 1, o).start()

                write(c, b).start()
            return carry

        lax.fori_loop(0, NGRP, body, 0)
        write(NCHUNK - 2, 0).wait()
        write(NCHUNK - 1, 1).wait()

    return k(x_r, table)


def kernel(x, table):
    x_r = x.reshape(NW, NCHUNK, R)
    out = _sc_gather(x_r, table)
    return out.reshape(BATCH, SEQ, D_MODEL)


# speedup vs baseline: 2.3549x; 1.0236x over previous
import functools

import jax
import jax.numpy as jnp
from jax import lax
from jax.experimental import pallas as pl
from jax.experimental.pallas import tpu as pltpu
from jax.experimental.pallas import tpu_sc as plsc

N_POSITION = 8192
D_MODEL = 1024
BATCH = 4
SEQ = 8192

NC, NS = 2, 16
NW = NC * NS
B_TOTAL = BATCH * SEQ
BPW = B_TOTAL // NW
R = 32
NCHUNK = BPW // R
NBUF = 3
NGRP = 10


@jax.jit
def _sc_gather(x_r, table):
    mesh = plsc.VectorSubcoreMesh(core_axis_name="c", subcore_axis_name="s")

    @functools.partial(
        pl.kernel,
        mesh=mesh,
        out_type=jax.ShapeDtypeStruct((B_TOTAL, D_MODEL), jnp.float32),
        scratch_types=[
            pltpu.VMEM((NCHUNK, R), jnp.int32),
            pltpu.VMEM((R, D_MODEL), jnp.float32),
            pltpu.VMEM((R, D_MODEL), jnp.float32),
            pltpu.VMEM((R, D_MODEL), jnp.float32),
            pltpu.SemaphoreType.DMA,
            pltpu.SemaphoreType.DMA,
            pltpu.SemaphoreType.DMA,
            pltpu.SemaphoreType.DMA,
            pltpu.SemaphoreType.DMA,
            pltpu.SemaphoreType.DMA,
        ],
    )
    def k(x_hbm, table_hbm, out_hbm, idx_v, rows0, rows1, rows2,
          gsem0, gsem1, gsem2, wsem0, wsem1, wsem2):
        wid = lax.axis_index("s") * NC + lax.axis_index("c")
        base = wid * BPW
        pltpu.sync_copy(x_hbm.at[wid], idx_v)

        bufs = (rows0, rows1, rows2)
        gsems = (gsem0, gsem1, gsem2)
        wsems = (wsem0, wsem1, wsem2)

        def gather(c, b):
            return pltpu.make_async_copy(
                table_hbm.at[idx_v.at[c]], bufs[b], gsems[b])

        def write(c, b):
            return pltpu.make_async_copy(
                bufs[b], out_hbm.at[pl.ds(base + c * R, R)], wsems[b])

        gather(0, 0).start()
        gather(1, 1).start()

        def body(g, carry):
            for j in range(NBUF):
                c = NBUF * g + j
                p = (j + NBUF - 1) % NBUF
                gather(c, j).wait()
                write(c, j).start()

                @pl.when(c >= 1)
                def _():
                    write(c - 1, p).wait()
                gather(c + 2, p).start()
            return carry

        lax.fori_loop(0, NGRP, body, 0)

        gather(NCHUNK - 2, 0).wait()
        write(NCHUNK - 2, 0).start()
        write(NCHUNK - 3, 2).wait()
        gather(NCHUNK - 1, 1).wait()
        write(NCHUNK - 1, 1).start()
        write(NCHUNK - 2, 0).wait()
        write(NCHUNK - 1, 1).wait()

    return k(x_r, table)


def kernel(x, table):
    x_r = x.reshape(NW, NCHUNK, R)
    out = _sc_gather(x_r, table)
    return out.reshape(BATCH, SEQ, D_MODEL)


# speedup vs baseline: 2.3650x; 1.0043x over previous
import functools

import jax
import jax.numpy as jnp
from jax import lax
from jax.experimental import pallas as pl
from jax.experimental.pallas import tpu as pltpu
from jax.experimental.pallas import tpu_sc as plsc

N_POSITION = 8192
D_MODEL = 1024
BATCH = 4
SEQ = 8192

NC, NS = 2, 16
NW = NC * NS
B_TOTAL = BATCH * SEQ
BPW = B_TOTAL // NW
R = 16
NCHUNK = BPW // R
NBUF = 4
NGRP = 15


@jax.jit
def _sc_gather(x_r, table):
    mesh = plsc.VectorSubcoreMesh(core_axis_name="c", subcore_axis_name="s")

    @functools.partial(
        pl.kernel,
        mesh=mesh,
        out_type=jax.ShapeDtypeStruct((B_TOTAL, D_MODEL), jnp.float32),
        scratch_types=[
            pltpu.VMEM((NCHUNK, R), jnp.int32),
            pltpu.VMEM((NBUF, R, D_MODEL), jnp.float32),
            pltpu.SemaphoreType.DMA,
            pltpu.SemaphoreType.DMA,
            pltpu.SemaphoreType.DMA,
            pltpu.SemaphoreType.DMA,
            pltpu.SemaphoreType.DMA,
            pltpu.SemaphoreType.DMA,
            pltpu.SemaphoreType.DMA,
            pltpu.SemaphoreType.DMA,
        ],
    )
    def k(x_hbm, table_hbm, out_hbm, idx_v, bufs,
          gsem0, gsem1, gsem2, gsem3, wsem0, wsem1, wsem2, wsem3):
        wid = lax.axis_index("s") * NC + lax.axis_index("c")
        base = wid * BPW
        pltpu.sync_copy(x_hbm.at[wid], idx_v)

        gsems = (gsem0, gsem1, gsem2, gsem3)
        wsems = (wsem0, wsem1, wsem2, wsem3)

        def gather(c, b):
            return pltpu.make_async_copy(
                table_hbm.at[idx_v.at[c]], bufs.at[b], gsems[b])

        def write(c, b):
            return pltpu.make_async_copy(
                bufs.at[b], out_hbm.at[pl.ds(base + c * R, R)], wsems[b])

        gather(0, 0).start()
        gather(1, 1).start()
        gather(2, 2).start()

        def body(g, carry):
            for j in range(NBUF):
                c = NBUF * g + j
                p = (j + NBUF - 1) % NBUF
                gather(c, j).wait()
                write(c, j).start()

                @pl.when(c >= 1)
                def _():
                    write(c - 1, p).wait()
                gather(c + 3, p).start()
            return carry

        lax.fori_loop(0, NGRP, body, 0)

        for c in range(NCHUNK - 4, NCHUNK):
            b = c % NBUF
            gather(c, b).wait()
            write(c, b).start()
            write(c - 1, (b + NBUF - 1) % NBUF).wait()
            if c == NCHUNK - 4:
                gather(NCHUNK - 1, (NCHUNK - 1) % NBUF).start()
        write(NCHUNK - 1, (NCHUNK - 1) % NBUF).wait()

    return k(x_r, table)


def kernel(x, table):
    x_r = x.reshape(NW, NCHUNK, R)
    out = _sc_gather(x_r, table)
    return out.reshape(BATCH, SEQ, D_MODEL)
